# baseline (speedup 1.0000x reference)
import functools

import jax
import jax.numpy as jnp
from jax import lax
from jax.experimental import pallas as pl
from jax.experimental.pallas import tpu as pltpu
from jax.experimental.pallas import tpu_sc as plsc

N = 10000
E = 320000
D = 128
H = 128
C = 2
G = 128

NC = 2
NS = 16
DH = D // NC
K = 128
CH = 160
SEG = 32
NSEG = CH // SEG
EP = NS * CH * K
NP = 10240
RPT = NP // NS
R = 512
NBLK = NP // R

_SPMEM_SLOT = (
    (True, False),
    (True, False),
    (True, False),
    (True, False),
    (False, False),
)


def _sc_body(xs_hbm, src_hbm, dst_hbm, agg_hbm, deg_hbm,
             srcv, dstv, msg, dsrc, xtab, agg_s, deg_s,
             semg0, semg1, semi, semj):
    c = lax.axis_index("c")
    s = lax.axis_index("s")
    base = s * RPT

    for k in range(RPT // K):
        pltpu.sync_copy(xs_hbm.at[pl.ds(c * NP + base + k * K, K)], msg.at[0])
        pltpu.sync_copy(msg.at[0], xtab.at[pl.ds(base + k * K, K)])

    zero16 = jnp.zeros((16,), jnp.float32)
    sixteenth16 = jnp.full((16,), 1.0 / 16.0, jnp.float32)

    def _zero_msg_row(r, carry):
        for k in range(DH // 16):
            msg[0, r, pl.ds(k * 16, 16)] = zero16
        return carry
    lax.fori_loop(0, K, _zero_msg_row, 0)

    def _zero_dsrc(r, carry):
        dsrc[r, pl.ds(0, 16)] = zero16
        return carry
    lax.fori_loop(0, K, _zero_dsrc, 0)

    for k in range(RPT // K):
        pltpu.sync_copy(msg.at[0], agg_s.at[pl.ds(base + k * K, K)])
        pltpu.sync_copy(dsrc, deg_s.at[pl.ds(base + k * K, K)])

    def _fill_dsrc(r, carry):
        dsrc[r, pl.ds(0, 16)] = sixteenth16
        return carry
    lax.fori_loop(0, K, _fill_dsrc, 0)

    pltpu.sync_copy(src_hbm.at[c, s, pl.ds(0, SEG)], srcv.at[0])
    pltpu.sync_copy(dst_hbm.at[s, pl.ds(0, SEG)], dstv.at[0])

    plsc.subcore_barrier()

    semg = (semg0, semg1)

    def _pair(tabs, qb, jj, issue_next):
        for b in range(2):
            j = jj * 2 + b
            tab = tabs[b]
            pltpu.make_async_copy(
                tab.at[srcv.at[qb, j]], msg.at[b], semg[b]).wait()
            pltpu.sync_copy(msg.at[b], agg_s.at[dstv.at[qb, j]], add=True)
            if issue_next:
                pltpu.async_copy(tab.at[srcv.at[qb, j + 2]], msg.at[b],
                                 semg[b])

            @pl.when(c == b)
            def _():
                pltpu.sync_copy(dsrc, deg_s.at[dstv.at[qb, j]], add=True)

    for q in range(NSEG):
        qb = q % 2
        tabs = tuple(xtab if sp else xs_hbm for sp in _SPMEM_SLOT[q])
        pltpu.async_copy(tabs[0].at[srcv.at[qb, 0]], msg.at[0], semg0)
        pltpu.async_copy(tabs[1].at[srcv.at[qb, 1]], msg.at[1], semg1)
        if q + 1 < NSEG:
            pltpu.async_copy(src_hbm.at[c, s, pl.ds((q + 1) * SEG, SEG)],
                             srcv.at[1 - qb], semi)
            pltpu.async_copy(dst_hbm.at[s, pl.ds((q + 1) * SEG, SEG)],
                             dstv.at[1 - qb], semj)

        def _lp(jj, carry, tabs=tabs, qb=qb):
            _pair(tabs, qb, jj, True)
            return carry
        lax.fori_loop(0, SEG // 2 - 1, _lp, 0)
        _pair(tabs, qb, SEG // 2 - 1, False)

        if q + 1 < NSEG:
            pltpu.make_async_copy(
                src_hbm.at[c, s, pl.ds((q + 1) * SEG, SEG)],
                srcv.at[1 - qb], semi).wait()
            pltpu.make_async_copy(
                dst_hbm.at[s, pl.ds((q + 1) * SEG, SEG)],
                dstv.at[1 - qb], semj).wait()

    plsc.subcore_barrier()

    pltpu.sync_copy(agg_s.at[pl.ds(base, RPT)],
                    agg_hbm.at[c, pl.ds(base, RPT)])
    pltpu.sync_copy(deg_s.at[pl.ds(base, RPT)],
                    deg_hbm.at[c, pl.ds(base, RPT)])


@functools.cache
def _sc_edge_agg():
    return pl.kernel(
        _sc_body,
        mesh=plsc.VectorSubcoreMesh(core_axis_name="c", subcore_axis_name="s"),
        compiler_params=pltpu.CompilerParams(use_tc_tiling_on_sc=False),
        out_type=(
            jax.ShapeDtypeStruct((NC, NP, DH), jnp.float32),
            jax.ShapeDtypeStruct((NC, NP, 16), jnp.float32),
        ),
        scratch_types=(
            pltpu.VMEM((2, SEG, K), jnp.int32),
            pltpu.VMEM((2, SEG, K), jnp.int32),
            pltpu.VMEM((2, K, DH), jnp.float32),
            pltpu.VMEM((K, 16), jnp.float32),
            pltpu.VMEM_SHARED((NP, DH), jnp.float32),
            pltpu.VMEM_SHARED((NP, DH), jnp.float32),
            pltpu.VMEM_SHARED((NP, 16), jnp.float32),
            pltpu.SemaphoreType.DMA,
            pltpu.SemaphoreType.DMA,
            pltpu.SemaphoreType.DMA,
            pltpu.SemaphoreType.DMA,
        ),
    )


def _tc_body(agg_ref, deg_ref, x_ref, batch_ref, wl_ref, wr_ref, bl_ref,
             w2_ref, b2_ref, out_ref, acc_ref):
    i = pl.program_id(0)

    @pl.when(i == 0)
    def _():
        acc_ref[...] = jnp.full((G, H), -jnp.inf, jnp.float32)

    deg = jnp.sum(deg_ref[0] + deg_ref[1], axis=1, keepdims=True)
    inv = 1.0 / jnp.maximum(deg, 1.0)
    mean0 = agg_ref[0] * inv
    mean1 = agg_ref[1] * inv
    hpre = (
        jnp.dot(mean0, wl_ref[pl.ds(0, DH), :],
                preferred_element_type=jnp.float32,
                precision=lax.Precision.HIGHEST)
        + jnp.dot(mean1, wl_ref[pl.ds(DH, DH), :],
                  preferred_element_type=jnp.float32,
                  precision=lax.Precision.HIGHEST)
        + jnp.dot(x_ref[...], wr_ref[...], preferred_element_type=jnp.float32,
                  precision=lax.Precision.HIGHEST)
        + bl_ref[...]
    )
    h = jnp.maximum(hpre, 0.0)

    b = batch_ref[0]
    rows = i * R + lax.broadcasted_iota(jnp.int32, (R, 1), 0)
    valid = rows < N
    neg = jnp.float32(-jnp.inf)
    glo = jnp.min(b)
    ghi = jnp.max(b)

    def _seg(g, carry):
        m = jnp.logical_and(b == g, valid)
        part = jnp.max(jnp.where(m, h, neg), axis=0, keepdims=True)
        acc_ref[pl.ds(g, 1), :] = jnp.maximum(acc_ref[pl.ds(g, 1), :], part)
        return carry
    lax.fori_loop(glo, ghi + 1, _seg, 0)

    @pl.when(i == NBLK - 1)
    def _():
        pooled = acc_ref[...]
        logits = lax.dot_general(
            pooled, w2_ref[...], (((1,), (1,)), ((), ())),
            preferred_element_type=jnp.float32,
            precision=lax.Precision.HIGHEST,
        ) + b2_ref[...]
        mx = jnp.max(logits, axis=1, keepdims=True)
        lse = jnp.log(jnp.sum(jnp.exp(logits - mx), axis=1, keepdims=True)) + mx
        out_ref[...] = logits - lse


_tc_head = pl.pallas_call(
    _tc_body,
    grid=(NBLK,),
    in_specs=[
        pl.BlockSpec((NC, R, DH), lambda i: (0, i, 0)),
        pl.BlockSpec((NC, R, 16), lambda i: (0, i, 0)),
        pl.BlockSpec((R, D), lambda i: (i, 0)),
        pl.BlockSpec((1, R, 1), lambda i: (i, 0, 0)),
        pl.BlockSpec((D, H), lambda i: (0, 0)),
        pl.BlockSpec((D, H), lambda i: (0, 0)),
        pl.BlockSpec((1, H), lambda i: (0, 0)),
        pl.BlockSpec((C, H), lambda i: (0, 0)),
        pl.BlockSpec((1, C), lambda i: (0, 0)),
    ],
    out_specs=pl.BlockSpec((G, C), lambda i: (0, 0)),
    out_shape=jax.ShapeDtypeStruct((G, C), jnp.float32),
    scratch_shapes=[pltpu.VMEM((G, H), jnp.float32)],
)


def kernel(x, edge_index, batch, W_l, b_l, W_r, W2, b2):
    src = edge_index[0]
    dst = edge_index[1]
    pad = EP - E
    src1 = jnp.concatenate(
        [src, jnp.zeros((pad,), jnp.int32)]).reshape(NS, CH, K)
    hbm_chunk = jnp.asarray(
        [0 if _SPMEM_SLOT[j // SEG][j % 2] else 1 for j in range(CH)],
        dtype=jnp.int32)[None, None, :, None]
    coff = NP * jnp.arange(NC, dtype=jnp.int32)[:, None, None, None]
    srcp = src1[None] + hbm_chunk * coff
    dstp = jnp.concatenate(
        [dst, jnp.full((pad,), N, jnp.int32)]).reshape(NS, CH, K)
    xp = jnp.pad(x, ((0, NP - N), (0, 0)))
    xs = jnp.concatenate([xp[:, :DH], xp[:, DH:]], axis=0)
    agg, deg = _sc_edge_agg()(xs, srcp, dstp)
    batch3 = jnp.concatenate(
        [batch, jnp.full((NP - N,), G - 1, jnp.int32)]).reshape(NBLK, R, 1)
    return _tc_head(agg, deg, xp, batch3, W_l.T, W_r.T,
                    b_l.reshape(1, H), W2, b2.reshape(1, C))


# speedup vs baseline: 1.1250x; 1.1250x over previous
import functools

import jax
import jax.numpy as jnp
from jax import lax
from jax.experimental import pallas as pl
from jax.experimental.pallas import tpu as pltpu
from jax.experimental.pallas import tpu_sc as plsc

N = 10000
E = 320000
D = 128
H = 128
C = 2
G = 128

NC = 2
NS = 16
DH = D // NC
K = 128
CH = 160
SEG = 32
NSEG = CH // SEG
EP = NS * CH * K
NP = 10240
RPT = NP // NS
R = 512
NBLK = NP // R

_SPMEM_SLOT = (
    (True, False),
    (True, False),
    (False, False),
    (True, False),
    (True, False),
)


def _sc_body(xs_hbm, src_hbm, dst_hbm, agg_hbm, deg_hbm,
             srcv, dstv, msg, dsrc, xtab, agg_s, deg_s,
             semg0, semg1, semi, semj):
    c = lax.axis_index("c")
    s = lax.axis_index("s")
    base = s * RPT

    for k in range(RPT // K):
        pltpu.sync_copy(xs_hbm.at[pl.ds(c * NP + base + k * K, K)], msg.at[0])
        pltpu.sync_copy(msg.at[0], xtab.at[pl.ds(base + k * K, K)])

    zero16 = jnp.zeros((16,), jnp.float32)
    sixteenth16 = jnp.full((16,), 1.0 / 16.0, jnp.float32)

    def _zero_msg_row(r, carry):
        for k in range(DH // 16):
            msg[0, r, pl.ds(k * 16, 16)] = zero16
        return carry
    lax.fori_loop(0, K, _zero_msg_row, 0)

    def _zero_dsrc(r, carry):
        dsrc[r, pl.ds(0, 16)] = zero16
        return carry
    lax.fori_loop(0, K, _zero_dsrc, 0)

    for k in range(RPT // K):
        pltpu.sync_copy(msg.at[0], agg_s.at[pl.ds(base + k * K, K)])
        pltpu.sync_copy(dsrc, deg_s.at[pl.ds(base + k * K, K)])

    def _fill_dsrc(r, carry):
        dsrc[r, pl.ds(0, 16)] = sixteenth16
        return carry
    lax.fori_loop(0, K, _fill_dsrc, 0)

    pltpu.sync_copy(src_hbm.at[c, s, pl.ds(0, SEG)], srcv.at[0])
    pltpu.sync_copy(dst_hbm.at[s, pl.ds(0, SEG)], dstv.at[0])

    plsc.subcore_barrier()

    semg = (semg0, semg1)

    def _pair(tabs, qb, jj, issue_next):
        for b in range(2):
            j = jj * 2 + b
            tab = tabs[b]
            pltpu.make_async_copy(
                tab.at[srcv.at[qb, j]], msg.at[b], semg[b]).wait()
            pltpu.sync_copy(msg.at[b], agg_s.at[dstv.at[qb, j]], add=True)
            if issue_next:
                pltpu.async_copy(tab.at[srcv.at[qb, j + 2]], msg.at[b],
                                 semg[b])

            @pl.when(c == b)
            def _():
                pltpu.sync_copy(dsrc, deg_s.at[dstv.at[qb, j]], add=True)

    for q in range(NSEG):
        qb = q % 2
        tabs = tuple(xtab if sp else xs_hbm for sp in _SPMEM_SLOT[q])
        pltpu.async_copy(tabs[0].at[srcv.at[qb, 0]], msg.at[0], semg0)
        pltpu.async_copy(tabs[1].at[srcv.at[qb, 1]], msg.at[1], semg1)
        if q + 1 < NSEG:
            pltpu.async_copy(src_hbm.at[c, s, pl.ds((q + 1) * SEG, SEG)],
                             srcv.at[1 - qb], semi)
            pltpu.async_copy(dst_hbm.at[s, pl.ds((q + 1) * SEG, SEG)],
                             dstv.at[1 - qb], semj)

        def _lp(jj, carry, tabs=tabs, qb=qb):
            _pair(tabs, qb, jj, True)
            return carry
        lax.fori_loop(0, SEG // 2 - 1, _lp, 0)
        _pair(tabs, qb, SEG // 2 - 1, False)

        if q + 1 < NSEG:
            pltpu.make_async_copy(
                src_hbm.at[c, s, pl.ds((q + 1) * SEG, SEG)],
                srcv.at[1 - qb], semi).wait()
            pltpu.make_async_copy(
                dst_hbm.at[s, pl.ds((q + 1) * SEG, SEG)],
                dstv.at[1 - qb], semj).wait()

    plsc.subcore_barrier()

    pltpu.sync_copy(agg_s.at[pl.ds(base, RPT)],
                    agg_hbm.at[c, pl.ds(base, RPT)])
    pltpu.sync_copy(deg_s.at[pl.ds(base, RPT)],
                    deg_hbm.at[c, pl.ds(base, RPT)])


@functools.cache
def _sc_edge_agg():
    return pl.kernel(
        _sc_body,
        mesh=plsc.VectorSubcoreMesh(core_axis_name="c", subcore_axis_name="s"),
        compiler_params=pltpu.CompilerParams(use_tc_tiling_on_sc=False),
        out_type=(
            jax.ShapeDtypeStruct((NC, NP, DH), jnp.float32),
            jax.ShapeDtypeStruct((NC, NP, 16), jnp.float32),
        ),
        scratch_types=(
            pltpu.VMEM((2, SEG, K), jnp.int32),
            pltpu.VMEM((2, SEG, K), jnp.int32),
            pltpu.VMEM((2, K, DH), jnp.float32),
            pltpu.VMEM((K, 16), jnp.float32),
            pltpu.VMEM_SHARED((NP, DH), jnp.float32),
            pltpu.VMEM_SHARED((NP, DH), jnp.float32),
            pltpu.VMEM_SHARED((NP, 16), jnp.float32),
            pltpu.SemaphoreType.DMA,
            pltpu.SemaphoreType.DMA,
            pltpu.SemaphoreType.DMA,
            pltpu.SemaphoreType.DMA,
        ),
    )


def _tc_body(agg_ref, deg_ref, x_ref, batch_ref, wl_ref, wr_ref, bl_ref,
             w2_ref, b2_ref, out_ref, acc_ref):
    i = pl.program_id(0)

    @pl.when(i == 0)
    def _():
        acc_ref[...] = jnp.full((G, H), -jnp.inf, jnp.float32)

    deg = jnp.sum(deg_ref[0] + deg_ref[1], axis=1, keepdims=True)
    inv = 1.0 / jnp.maximum(deg, 1.0)
    mean0 = agg_ref[0] * inv
    mean1 = agg_ref[1] * inv
    hpre = (
        jnp.dot(mean0, wl_ref[pl.ds(0, DH), :],
                preferred_element_type=jnp.float32,
                precision=lax.Precision.HIGHEST)
        + jnp.dot(mean1, wl_ref[pl.ds(DH, DH), :],
                  preferred_element_type=jnp.float32,
                  precision=lax.Precision.HIGHEST)
        + jnp.dot(x_ref[...], wr_ref[...], preferred_element_type=jnp.float32,
                  precision=lax.Precision.HIGHEST)
        + bl_ref[...]
    )
    h = jnp.maximum(hpre, 0.0)

    b = batch_ref[0]
    rows = i * R + lax.broadcasted_iota(jnp.int32, (R, 1), 0)
    valid = rows < N
    neg = jnp.float32(-jnp.inf)
    glo = jnp.min(b)
    ghi = jnp.max(b)

    def _seg(g, carry):
        m = jnp.logical_and(b == g, valid)
        part = jnp.max(jnp.where(m, h, neg), axis=0, keepdims=True)
        acc_ref[pl.ds(g, 1), :] = jnp.maximum(acc_ref[pl.ds(g, 1), :], part)
        return carry
    lax.fori_loop(glo, ghi + 1, _seg, 0)

    @pl.when(i == NBLK - 1)
    def _():
        pooled = acc_ref[...]
        logits = lax.dot_general(
            pooled, w2_ref[...], (((1,), (1,)), ((), ())),
            preferred_element_type=jnp.float32,
            precision=lax.Precision.HIGHEST,
        ) + b2_ref[...]
        mx = jnp.max(logits, axis=1, keepdims=True)
        lse = jnp.log(jnp.sum(jnp.exp(logits - mx), axis=1, keepdims=True)) + mx
        out_ref[...] = logits - lse


_tc_head = pl.pallas_call(
    _tc_body,
    grid=(NBLK,),
    in_specs=[
        pl.BlockSpec((NC, R, DH), lambda i: (0, i, 0)),
        pl.BlockSpec((NC, R, 16), lambda i: (0, i, 0)),
        pl.BlockSpec((R, D), lambda i: (i, 0)),
        pl.BlockSpec((1, R, 1), lambda i: (i, 0, 0)),
        pl.BlockSpec((D, H), lambda i: (0, 0)),
        pl.BlockSpec((D, H), lambda i: (0, 0)),
        pl.BlockSpec((1, H), lambda i: (0, 0)),
        pl.BlockSpec((C, H), lambda i: (0, 0)),
        pl.BlockSpec((1, C), lambda i: (0, 0)),
    ],
    out_specs=pl.BlockSpec((G, C), lambda i: (0, 0)),
    out_shape=jax.ShapeDtypeStruct((G, C), jnp.float32),
    scratch_shapes=[pltpu.VMEM((G, H), jnp.float32)],
)


def kernel(x, edge_index, batch, W_l, b_l, W_r, W2, b2):
    src = edge_index[0]
    dst = edge_index[1]
    pad = EP - E
    src1 = jnp.concatenate(
        [src, jnp.zeros((pad,), jnp.int32)]).reshape(NS, CH, K)
    hbm_chunk = jnp.asarray(
        [0 if _SPMEM_SLOT[j // SEG][j % 2] else 1 for j in range(CH)],
        dtype=jnp.int32)[None, None, :, None]
    coff = NP * jnp.arange(NC, dtype=jnp.int32)[:, None, None, None]
    srcp = src1[None] + hbm_chunk * coff
    dstp = jnp.concatenate(
        [dst, jnp.full((pad,), N, jnp.int32)]).reshape(NS, CH, K)
    xp = jnp.pad(x, ((0, NP - N), (0, 0)))
    xs = jnp.concatenate([xp[:, :DH], xp[:, DH:]], axis=0)
    agg, deg = _sc_edge_agg()(xs, srcp, dstp)
    batch3 = jnp.concatenate(
        [batch, jnp.full((NP - N,), G - 1, jnp.int32)]).reshape(NBLK, R, 1)
    return _tc_head(agg, deg, xp, batch3, W_l.T, W_r.T,
                    b_l.reshape(1, H), W2, b2.reshape(1, C))
